# baseline (speedup 1.0000x reference)
import functools

import jax
import jax.numpy as jnp
from jax import lax
from jax.experimental import pallas as pl
from jax.experimental.pallas import tpu as pltpu
from jax.experimental.pallas import tpu_sc as plsc

B = 16
S = 224 * 224
D = 40
NC = 1
NS = 16
NW = NC * NS
ROWS_TC = 192
ROWS_SC = 224 - ROWS_TC
S_TC = ROWS_TC * 224
S_SC = ROWS_SC * 224
CHUNK = S_SC // NW
PROWS = 56


def _sc_encode_body(tm_hbm, pos_hbm, part_hbm, mv, pv, av, sem):
    wid = lax.axis_index("s") * NC + lax.axis_index("c")
    base = wid * CHUNK
    c1 = pltpu.async_copy(tm_hbm.at[pl.ds(base, CHUNK)], mv, sem)
    c2 = pltpu.async_copy(
        pos_hbm.at[pl.ds((S_TC + base) * D, CHUNK * D)], pv, sem)
    c1.wait()
    c2.wait()

    zeros = jnp.zeros((16,), jnp.float32)
    NBLK = CHUNK // 16

    for grp in range(4):
        bs = list(range(grp * 4, grp * 4 + 4))

        def body(blk, acc, bs=bs):
            mvec = mv[pl.ds(blk * 16, 16)]
            out = list(acc)
            for j in range(16):
                s_off = (blk * 16 + j) * D
                p0 = pv[pl.ds(s_off, 16)]
                p1 = pv[pl.ds(s_off + 16, 16)]
                p2 = pv[pl.ds(s_off + 24, 16)]
                mj = mvec[j]
                for k, b in enumerate(bs):
                    c = (lax.shift_right_logical(mj, b) & 1) == 1
                    q = 3 * k
                    out[q] = jnp.where(c, out[q] + p0, out[q])
                    out[q + 1] = jnp.where(c, out[q + 1] + p1, out[q + 1])
                    out[q + 2] = jnp.where(c, out[q + 2] + p2, out[q + 2])
            return tuple(out)

        acc = lax.fori_loop(0, NBLK, body, (zeros,) * 12, unroll=False)
        for k in range(12):
            av[pl.ds((grp * 12 + k) * 16, 16)] = acc[k]

    def pbody(j, acc5):
        return tuple(acc5[k] + pv[pl.ds(j * 80 + k * 16, 16)]
                     for k in range(5))

    acc5 = lax.fori_loop(0, CHUNK * D // 80, pbody, (zeros,) * 5,
                         unroll=False)
    for k in range(5):
        av[pl.ds((3 * B + k) * 16, 16)] = acc5[k]
    for r in range(3 * B + 5, PROWS):
        av[pl.ds(r * 16, 16)] = zeros

    pltpu.sync_copy(av, part_hbm.at[wid])


def _make_sc_encode():
    mesh = plsc.VectorSubcoreMesh(core_axis_name="c", subcore_axis_name="s", num_cores=NC)
    return pl.kernel(
        _sc_encode_body,
        mesh=mesh,
        out_type=jax.ShapeDtypeStruct((NW, PROWS * 16), jnp.float32),
        scratch_types=[
            pltpu.VMEM((CHUNK,), jnp.int32),
            pltpu.VMEM((CHUNK * D,), jnp.float32),
            pltpu.VMEM((PROWS * 16,), jnp.float32),
            pltpu.SemaphoreType.DMA,
        ],
    )


def _tmask_kernel(x_ref, out_ref):
    pw = jnp.left_shift(
        jnp.ones((B, 1), jnp.int32),
        jax.lax.broadcasted_iota(jnp.int32, (B, 1), 0))
    for r in range(ROWS_SC):
        t = (x_ref[:, r, :] > 0.5).astype(jnp.int32)
        m = jnp.sum(t * pw, axis=0, keepdims=True)
        out_ref[pl.ds(r * 224, 224)] = m[0]


def _tmask(x):
    return pl.pallas_call(
        _tmask_kernel,
        grid=(1,),
        in_specs=[pl.BlockSpec((B, ROWS_SC, 224),
                               lambda i: (0, ROWS_TC // ROWS_SC, 0))],
        out_specs=pl.BlockSpec((S_SC,), lambda i: (0,)),
        out_shape=jax.ShapeDtypeStruct((S_SC,), jnp.int32),
    )(x)


def _tc_partial_kernel(x_ref, pos_ref, out_ref):
    step = pl.program_id(0)

    @pl.when(step == 0)
    def _():
        out_ref[...] = jnp.zeros_like(out_ref)

    acc = out_ref[...]
    for r in range(8):
        t = jnp.where(x_ref[:, r, :] > 0.5, 1.0, 0.0)
        t17 = jnp.concatenate([t, jnp.ones((1, 224), jnp.float32)], axis=0)
        acc = acc + jnp.dot(t17, pos_ref[r],
                            preferred_element_type=jnp.float32)
    out_ref[...] = acc


def _tc_partial(x, pos3):
    nsteps = ROWS_TC // 8
    return pl.pallas_call(
        _tc_partial_kernel,
        grid=(nsteps,),
        in_specs=[
            pl.BlockSpec((B, 8, 224), lambda i: (0, i, 0)),
            pl.BlockSpec((8, 224, D), lambda i: (i, 0, 0)),
        ],
        out_specs=pl.BlockSpec((B + 1, D), lambda i: (0, 0)),
        out_shape=jax.ShapeDtypeStruct((B + 1, D), jnp.float32),
    )(x, pos3)


def _combine_kernel(atc_ref, pa_ref, pp_ref, vw_ref, cw_ref, out_ref):
    a48 = jnp.sum(pa_ref[...], axis=0)
    A_sc = jnp.concatenate([a48[:, :32], a48[:, 40:48]], axis=1)
    p80 = jnp.sum(pp_ref[...], axis=0)
    A = A_sc + atc_ref[:B, :]
    p40 = p80[:, :D] + p80[:, D:] + atc_ref[B:B + 1, :]
    v0 = vw_ref[0:1, :]
    v1 = vw_ref[1:2, :]
    sample = v0 * (p40 - A) + v1 * A
    enc = jnp.where(sample > 0, 1.0, -1.0)
    out_ref[...] = lax.dot_general(
        enc, cw_ref[...], (((1,), (1,)), ((), ())),
        preferred_element_type=jnp.float32)


def kernel(x, position_weight, value_weight, classify_weight):
    pos_lin = position_weight.reshape(-1)
    tm = _tmask(x)
    pos3 = position_weight.reshape(28, 8 * 224, D)[:ROWS_TC // 8]
    pos3 = pos3.reshape(ROWS_TC // 8 * 8, 224, D)
    part = _make_sc_encode()(tm, pos_lin)
    atc = _tc_partial(x, pos3)
    part_a = part[:, :3 * B * 16].reshape(NW, B, 48)
    part_p = part[:, 3 * B * 16:(3 * B + 5) * 16].reshape(NW, 1, 80)
    return pl.pallas_call(
        _combine_kernel,
        out_shape=jax.ShapeDtypeStruct((B, classify_weight.shape[0]),
                                       jnp.float32),
    )(atc, part_a, part_p, value_weight, classify_weight)


# speedup vs baseline: 1.3807x; 1.3807x over previous
import functools

import jax
import jax.numpy as jnp
from jax import lax
from jax.experimental import pallas as pl
from jax.experimental.pallas import tpu as pltpu
from jax.experimental.pallas import tpu_sc as plsc

B = 16
S = 224 * 224
D = 40
NC = 1
NS = 16
NW = NC * NS
ROWS_TC = 192
ROWS_SC = 224 - ROWS_TC
S_TC = ROWS_TC * 224
S_SC = ROWS_SC * 224
CHUNK = S_SC // NW
PROWS = 56


def _sc_encode_body(tm_hbm, pos_hbm, part_hbm, mv, pv, av, sem):
    wid = lax.axis_index("s") * NC + lax.axis_index("c")
    base = wid * CHUNK
    c1 = pltpu.async_copy(tm_hbm.at[pl.ds(base, CHUNK)], mv, sem)
    c2 = pltpu.async_copy(pos_hbm.at[pl.ds(base * D, CHUNK * D)], pv, sem)
    c1.wait()
    c2.wait()

    zeros = jnp.zeros((16,), jnp.float32)
    NBLK = CHUNK // 16

    for grp in range(4):
        bs = list(range(grp * 4, grp * 4 + 4))

        def body(blk, acc, bs=bs):
            mvec = mv[pl.ds(blk * 16, 16)]
            out = list(acc)
            for j in range(16):
                s_off = (blk * 16 + j) * D
                p0 = pv[pl.ds(s_off, 16)]
                p1 = pv[pl.ds(s_off + 16, 16)]
                p2 = pv[pl.ds(s_off + 24, 16)]
                mj = mvec[j]
                for k, b in enumerate(bs):
                    c = (lax.shift_right_logical(mj, b) & 1) == 1
                    q = 3 * k
                    out[q] = jnp.where(c, out[q] + p0, out[q])
                    out[q + 1] = jnp.where(c, out[q + 1] + p1, out[q + 1])
                    out[q + 2] = jnp.where(c, out[q + 2] + p2, out[q + 2])
            return tuple(out)

        acc = lax.fori_loop(0, NBLK, body, (zeros,) * 12, unroll=False)
        for k in range(12):
            av[pl.ds((grp * 12 + k) * 16, 16)] = acc[k]

    def pbody(j, acc5):
        return tuple(acc5[k] + pv[pl.ds(j * 80 + k * 16, 16)]
                     for k in range(5))

    acc5 = lax.fori_loop(0, CHUNK * D // 80, pbody, (zeros,) * 5,
                         unroll=False)
    for k in range(5):
        av[pl.ds((3 * B + k) * 16, 16)] = acc5[k]
    for r in range(3 * B + 5, PROWS):
        av[pl.ds(r * 16, 16)] = zeros

    pltpu.sync_copy(av, part_hbm.at[wid])


def _make_sc_encode():
    mesh = plsc.VectorSubcoreMesh(core_axis_name="c", subcore_axis_name="s", num_cores=NC)
    return pl.kernel(
        _sc_encode_body,
        mesh=mesh,
        out_type=jax.ShapeDtypeStruct((NW, PROWS * 16), jnp.float32),
        scratch_types=[
            pltpu.VMEM((CHUNK,), jnp.int32),
            pltpu.VMEM((CHUNK * D,), jnp.float32),
            pltpu.VMEM((PROWS * 16,), jnp.float32),
            pltpu.SemaphoreType.DMA,
        ],
    )


def _tmask_kernel(x_ref, out_ref):
    pw = jnp.left_shift(
        jnp.ones((B, 1), jnp.int32),
        jax.lax.broadcasted_iota(jnp.int32, (B, 1), 0))
    for r in range(ROWS_SC):
        t = (x_ref[:, r, :] > 0.5).astype(jnp.int32)
        m = jnp.sum(t * pw, axis=0, keepdims=True)
        out_ref[pl.ds(r * 224, 224)] = m[0]


def _tmask(x):
    return pl.pallas_call(
        _tmask_kernel,
        grid=(1,),
        in_specs=[pl.BlockSpec((B, ROWS_SC, 224),
                               lambda i: (0, ROWS_TC // ROWS_SC, 0))],
        out_specs=pl.BlockSpec((S_SC,), lambda i: (0,)),
        out_shape=jax.ShapeDtypeStruct((S_SC,), jnp.int32),
    )(x)


def _tc_partial_kernel(x_ref, pos_ref, out_ref):
    step = pl.program_id(0)

    @pl.when(step == 0)
    def _():
        out_ref[...] = jnp.zeros_like(out_ref)

    acc = out_ref[...]
    for r in range(8):
        t = jnp.where(x_ref[:, r, :] > 0.5, 1.0, 0.0)
        t17 = jnp.concatenate([t, jnp.ones((1, 224), jnp.float32)], axis=0)
        acc = acc + jnp.dot(t17, pos_ref[r],
                            preferred_element_type=jnp.float32)
    out_ref[...] = acc


def _tc_partial(x, pos3):
    nsteps = ROWS_TC // 8
    return pl.pallas_call(
        _tc_partial_kernel,
        grid=(nsteps,),
        in_specs=[
            pl.BlockSpec((B, 8, 224), lambda i: (0, i, 0)),
            pl.BlockSpec((8, 224, D), lambda i: (i, 0, 0)),
        ],
        out_specs=pl.BlockSpec((B + 1, D), lambda i: (0, 0)),
        out_shape=jax.ShapeDtypeStruct((B + 1, D), jnp.float32),
    )(x, pos3)


def _combine_kernel(atc_ref, pa_ref, pp_ref, vw_ref, cw_ref, out_ref):
    a48 = jnp.sum(pa_ref[...], axis=0)
    A_sc = jnp.concatenate([a48[:, :32], a48[:, 40:48]], axis=1)
    p80 = jnp.sum(pp_ref[...], axis=0)
    A = A_sc + atc_ref[:B, :]
    p40 = p80[:, :D] + p80[:, D:] + atc_ref[B:B + 1, :]
    v0 = vw_ref[0:1, :]
    v1 = vw_ref[1:2, :]
    sample = v0 * (p40 - A) + v1 * A
    enc = jnp.where(sample > 0, 1.0, -1.0)
    out_ref[...] = lax.dot_general(
        enc, cw_ref[...], (((1,), (1,)), ((), ())),
        preferred_element_type=jnp.float32)


def kernel(x, position_weight, value_weight, classify_weight):
    pos_sc = position_weight[S_TC:].reshape(-1)
    tm = _tmask(x)
    pos3 = position_weight.reshape(28, 8 * 224, D)[:ROWS_TC // 8]
    pos3 = pos3.reshape(ROWS_TC // 8 * 8, 224, D)
    part = _make_sc_encode()(tm, pos_sc)
    atc = _tc_partial(x, pos3)
    part_a = part[:, :3 * B * 16].reshape(NW, B, 48)
    part_p = part[:, 3 * B * 16:(3 * B + 5) * 16].reshape(NW, 1, 80)
    return pl.pallas_call(
        _combine_kernel,
        out_shape=jax.ShapeDtypeStruct((B, classify_weight.shape[0]),
                                       jnp.float32),
    )(atc, part_a, part_p, value_weight, classify_weight)
